# Initial kernel scaffold; baseline (speedup 1.0000x reference)
import jax
import jax.numpy as jnp
from jax.experimental import pallas as pl


def kernel(x, y, r, edge_index, t_edge_index, W_src, b_src, W_dst, b_dst, W_edge, b_edge, attn, W1, b1, W2, b2):
    raise NotImplementedError("write your pallas kernel here")


# speedup vs baseline: 20.3109x; 20.3109x over previous
import functools

import jax
import jax.numpy as jnp
from jax import lax
from jax.experimental import pallas as pl
from jax.experimental.pallas import tpu as pltpu
from jax.experimental.pallas import tpu_sc as plsc

NN = 10000
NE = 160000
NT = 640000
DD = 256
DM = 64
HH = 1024
XRW = 80

NC, NS, LL = 2, 16, 16
NW = NC * NS
BT = 80
NBT = NT // (NW * BT)
BE = 80
NBE = NE // BE

_mesh = plsc.VectorSubcoreMesh(
    core_axis_name="c", subcore_axis_name="s", num_cores=NC, num_subcores=NS)
_sc_params = pltpu.CompilerParams(needs_layout_passes=False,
                                  use_tc_tiling_on_sc=False)


def _worker_id():
    return lax.axis_index("s") * NC + lax.axis_index("c")


def _rsqrt3(n2):
    i = plsc.bitcast(n2, jnp.int32)
    y = plsc.bitcast(jnp.int32(0x5F3759DF) - (i >> 1), jnp.float32)
    for _ in range(3):
        y = y * (1.5 - 0.5 * n2 * y * y)
    return y


def _silu(v):
    return v / (1.0 + jnp.exp(-v))


def _build_body(xj, xi, yp, srcv, dstv, rv, xr_out,
                idx_s, idx_d, ypb, rb, gx, gy, xrb, s1, s2):
    wid = _worker_id()
    iota = lax.iota(jnp.int32, 16)
    zero16 = jnp.zeros((16,), jnp.float32)

    def _zpad(t, _):
        xrb[pl.ds(t * XRW + DM, 16)] = zero16
        return 0
    lax.fori_loop(0, BE, _zpad, 0)

    def _blk(i, _):
        b = wid + NW * i

        @pl.when(b < NBE)
        def _():
            base = b * BE
            pltpu.sync_copy(srcv.at[pl.ds(base, BE)], idx_s)
            pltpu.sync_copy(dstv.at[pl.ds(base, BE)], idx_d)
            pltpu.sync_copy(yp.at[pl.ds(base, BE)], ypb)
            pltpu.sync_copy(rv.at[pl.ds(base * 3, BE * 3)],
                            rb.at[pl.ds(0, BE * 3)])
            d1 = pltpu.async_copy(xj.at[idx_s], gx, s1)
            d2 = pltpu.async_copy(xi.at[idx_d], gy, s2)
            d1.wait()
            d2.wait()

            def _row(t, _2):
                for k in range(4):
                    sl = pl.ds(k * 16, 16)
                    xrb[pl.ds(t * XRW + k * 16, 16)] = (
                        gx[t, sl] + gy[t, sl] + ypb[t, sl])
                return 0
            lax.fori_loop(0, BE, _row, 0)

            for g in range(BE // 16):
                rows3 = (g * 16 + iota) * 3
                rows80 = (g * 16 + iota) * XRW
                rx = plsc.load_gather(rb, [rows3])
                ry = plsc.load_gather(rb, [rows3 + 1])
                rz = plsc.load_gather(rb, [rows3 + 2])
                inv = _rsqrt3(rx * rx + ry * ry + rz * rz)
                plsc.store_scatter(xrb, [rows80 + DM], rx * inv)
                plsc.store_scatter(xrb, [rows80 + DM + 1], ry * inv)
                plsc.store_scatter(xrb, [rows80 + DM + 2], rz * inv)
            pltpu.sync_copy(xrb, xr_out.at[pl.ds(base * XRW, BE * XRW)])
        return 0
    lax.fori_loop(0, (NBE + NW - 1) // NW, _blk, 0)


_build = pl.kernel(
    _build_body,
    out_type=jax.ShapeDtypeStruct((NE * XRW,), jnp.float32),
    mesh=_mesh,
    compiler_params=_sc_params,
    scratch_types=[
        pltpu.VMEM((BE,), jnp.int32), pltpu.VMEM((BE,), jnp.int32),
        pltpu.VMEM((BE, DM), jnp.float32), pltpu.VMEM((256,), jnp.float32),
        pltpu.VMEM((BE, DM), jnp.float32), pltpu.VMEM((BE, DM), jnp.float32),
        pltpu.VMEM((BE * XRW,), jnp.float32),
        pltpu.SemaphoreType.DMA, pltpu.SemaphoreType.DMA,
    ],
)


def _score_body(xr, tsv, tdv, attn4, a_out, tmax_out,
                its, itd, rts, rtd, zb, ab, atb, tmb, s1, s2):
    wid = _worker_id()
    iota = lax.iota(jnp.int32, 16)
    ones = jnp.ones((16,), jnp.float32)
    pltpu.sync_copy(attn4, atb)
    ninf = jnp.full((16,), -3.0e38, jnp.float32)

    def _blk(i, mv):
        base = wid * (NT // NW) + i * BT
        pltpu.sync_copy(tsv.at[pl.ds(base, BT)], its)
        pltpu.sync_copy(tdv.at[pl.ds(base, BT)], itd)
        d1 = pltpu.async_copy(xr.at[its], rts, s1)
        d2 = pltpu.async_copy(xr.at[itd], rtd, s2)
        d1.wait()
        d2.wait()

        def _grp(g, _):
            cosv = jnp.zeros((16,), jnp.float32)
            for k2 in range(16):
                t = g * 16 + k2
                us = rts[t, pl.ds(DM, 16)]
                ud = rtd[t, pl.ds(DM, 16)]
                cs = lax.broadcast(jnp.sum(us * ud), (16,))
                cosv = jnp.where(iota == k2, cs, cosv)
            two = cosv + cosv
            base32 = (g * 16 + iota) * 32
            z0, z1 = ones, cosv
            plsc.store_scatter(zb, [base32], z0)
            plsc.store_scatter(zb, [base32 + 1], z1)
            for c in range(2, 32):
                z0, z1 = z1, two * z1 - z0
                plsc.store_scatter(zb, [base32 + c], z1)
            return 0
        lax.fori_loop(0, BT // 16, _grp, 0)

        def _bt(g, mv):
            accv = jnp.zeros((16,), jnp.float32)
            for k2 in range(16):
                t = g * 16 + k2
                zc0 = zb[pl.ds(t * 32, 16)]
                zc1 = zb[pl.ds(t * 32 + 16, 16)]
                two16 = lax.broadcast(zc1[0], (16,))
                two16 = two16 + two16
                acc = jnp.zeros((16,), jnp.float32)
                zm2, zm1 = zc0, zc1
                for k in range(4):
                    if k == 0:
                        z = zc0
                    elif k == 1:
                        z = zc1
                    else:
                        z = two16 * zm1 - zm2
                        zm2, zm1 = zm1, z
                    x = rts[t, pl.ds(k * 16, 16)] + rtd[t, pl.ds(k * 16, 16)]
                    acc = acc + _silu(z + x) * atb[k]
                at = lax.broadcast(jnp.sum(acc), (16,))
                accv = jnp.where(iota == k2, at, accv)
            ab[pl.ds(g * 16, 16)] = accv
            return jnp.maximum(mv, accv)
        mv = lax.fori_loop(0, BT // 16, _bt, mv)
        pltpu.sync_copy(ab, a_out.at[pl.ds(base, BT)])
        return mv

    mv = lax.fori_loop(0, NBT, _blk, ninf)
    tmb[...] = mv
    pltpu.sync_copy(tmb, tmax_out.at[wid])


_score = pl.kernel(
    _score_body,
    out_type=(jax.ShapeDtypeStruct((NT,), jnp.float32),
              jax.ShapeDtypeStruct((NW, 16), jnp.float32)),
    mesh=_mesh,
    compiler_params=_sc_params,
    scratch_types=[
        pltpu.VMEM((BT,), jnp.int32), pltpu.VMEM((BT,), jnp.int32),
        pltpu.VMEM((BT, XRW), jnp.float32), pltpu.VMEM((BT, XRW), jnp.float32),
        pltpu.VMEM((BT * 32,), jnp.float32), pltpu.VMEM((BT,), jnp.float32),
        pltpu.VMEM((4, 16), jnp.float32), pltpu.VMEM((16,), jnp.float32),
        pltpu.SemaphoreType.DMA, pltpu.SemaphoreType.DMA,
    ],
)


def _denom_body(av, tdv, tmaxv, aexp_out, part_out, ab, itd, tmb, zb2, expb,
                asum_sh):
    cid = lax.axis_index("c")
    sid = lax.axis_index("s")
    wid = _worker_id()
    pltpu.sync_copy(tmaxv, tmb)
    mv = tmb[0]
    for w in range(1, NW):
        mv = jnp.maximum(mv, tmb[w])
    gm = jnp.max(mv)

    if True:
        zero16 = jnp.zeros((16,), jnp.float32)

        def _z(j, _):
            zb2[pl.ds(j * 16, 16)] = zero16
            return 0
        lax.fori_loop(0, 2000 // 16, _z, 0)
        for q in range(5):
            pltpu.sync_copy(
                zb2, asum_sh.at[pl.ds(sid * 10000 + q * 2000, 2000)])
        plsc.subcore_barrier()

        def _blk(i, _):
            base = wid * (NT // NW) + i * BT
            pltpu.sync_copy(av.at[pl.ds(base, BT)], ab)
            pltpu.sync_copy(tdv.at[pl.ds(base, BT)], itd)
            for g in range(BT // 16):
                sl = pl.ds(g * 16, 16)
                ab[sl] = jnp.exp(ab[sl] - gm)
            pltpu.sync_copy(ab, aexp_out.at[pl.ds(base, BT)])
            pltpu.sync_copy(ab, asum_sh.at[itd], add=True)
            return 0
        lax.fori_loop(0, NBT, _blk, 0)
        plsc.subcore_barrier()
        pltpu.sync_copy(asum_sh.at[pl.ds(sid * 10000, 10000)], expb)
        pltpu.sync_copy(expb, part_out.at[cid, pl.ds(sid * 10000, 10000)])


_denom = pl.kernel(
    _denom_body,
    out_type=(jax.ShapeDtypeStruct((NT,), jnp.float32),
              jax.ShapeDtypeStruct((NC, NE), jnp.float32)),
    mesh=_mesh,
    compiler_params=_sc_params,
    scratch_types=[
        pltpu.VMEM((BT,), jnp.float32), pltpu.VMEM((BT,), jnp.int32),
        pltpu.VMEM((NW, 16), jnp.float32), pltpu.VMEM((2000,), jnp.float32),
        pltpu.VMEM((10000,), jnp.float32),
        pltpu.VMEM_SHARED((NE,), jnp.float32),
    ],
)


def _inv_body(partv, ainv_out, b0, b1):
    wid = _worker_id()

    def _blk(i, _):
        b = wid + NW * i

        @pl.when(b < NBE)
        def _():
            base = b * BE
            pltpu.sync_copy(partv.at[0, pl.ds(base, BE)], b0)
            pltpu.sync_copy(partv.at[1, pl.ds(base, BE)], b1)
            for g in range(BE // 16):
                sl = pl.ds(g * 16, 16)
                b0[sl] = 1.0 / (b0[sl] + b1[sl] + 1e-9)
            pltpu.sync_copy(b0, ainv_out.at[pl.ds(base, BE)])
        return 0
    lax.fori_loop(0, (NBE + NW - 1) // NW, _blk, 0)


_inv = pl.kernel(
    _inv_body,
    out_type=jax.ShapeDtypeStruct((NE,), jnp.float32),
    mesh=_mesh,
    compiler_params=_sc_params,
    scratch_types=[
        pltpu.VMEM((BE,), jnp.float32), pltpu.VMEM((BE,), jnp.float32),
    ],
)


def _msg_body(xr, tsv, tdv, aev, ainvv, dstv, ftn_out,
              its, itd, aeb, avb, ndb, rts, msg, expb, s1, s2, s3, ftn_sh):
    cid = lax.axis_index("c")
    sid = lax.axis_index("s")
    wid = _worker_id()
    rows_per_tile = NN // NS

    if True:
        zero16 = jnp.zeros((16,), jnp.float32)

        def _z(t, _):
            for k in range(4):
                expb[t, pl.ds(k * 16, 16)] = zero16
            return 0
        lax.fori_loop(0, rows_per_tile, _z, 0)
        pltpu.sync_copy(expb, ftn_sh.at[pl.ds(sid * rows_per_tile,
                                              rows_per_tile)])
        plsc.subcore_barrier()

        def _blk(i, _):
            base = wid * (NT // NW) + i * BT
            pltpu.sync_copy(tsv.at[pl.ds(base, BT)], its)
            pltpu.sync_copy(tdv.at[pl.ds(base, BT)], itd)
            pltpu.sync_copy(aev.at[pl.ds(base, BT)], aeb)
            d1 = pltpu.async_copy(xr.at[its], rts, s1)
            d2 = pltpu.async_copy(ainvv.at[itd], avb, s2)
            d3 = pltpu.async_copy(dstv.at[itd], ndb, s3)
            d1.wait()
            d2.wait()
            d3.wait()
            for g in range(BT // 16):
                sl = pl.ds(g * 16, 16)
                aeb[sl] = aeb[sl] * avb[sl]

            def _row(g, _2):
                w16 = aeb[pl.ds(g * 16, 16)]
                for k2 in range(16):
                    t = g * 16 + k2
                    wt = lax.broadcast(w16[k2], (16,))
                    for k in range(4):
                        sl = pl.ds(k * 16, 16)
                        msg[t, sl] = rts[t, sl] * wt
                return 0
            lax.fori_loop(0, BT // 16, _row, 0)
            pltpu.sync_copy(msg, ftn_sh.at[ndb], add=True)
            return 0
        lax.fori_loop(0, NBT, _blk, 0)
        plsc.subcore_barrier()
        pltpu.sync_copy(ftn_sh.at[pl.ds(sid * rows_per_tile, rows_per_tile)],
                        expb)
        pltpu.sync_copy(expb, ftn_out.at[cid, pl.ds(sid * rows_per_tile,
                                                    rows_per_tile)])


_msg = pl.kernel(
    _msg_body,
    out_type=jax.ShapeDtypeStruct((NC, NN, DM), jnp.float32),
    mesh=_mesh,
    compiler_params=_sc_params,
    scratch_types=[
        pltpu.VMEM((BT,), jnp.int32), pltpu.VMEM((BT,), jnp.int32),
        pltpu.VMEM((BT,), jnp.float32), pltpu.VMEM((BT,), jnp.float32),
        pltpu.VMEM((BT,), jnp.int32),
        pltpu.VMEM((BT, XRW), jnp.float32), pltpu.VMEM((BT, DM), jnp.float32),
        pltpu.VMEM((NN // NS, DM), jnp.float32),
        pltpu.SemaphoreType.DMA, pltpu.SemaphoreType.DMA,
        pltpu.SemaphoreType.DMA,
        pltpu.VMEM_SHARED((NN, DM), jnp.float32),
    ],
)


def _proj2_body(x_ref, wa, ba, wb, bb, oa, ob):
    xv = x_ref[...]
    oa[...] = jnp.dot(xv, wa[...], preferred_element_type=jnp.float32) + ba[...]
    ob[...] = jnp.dot(xv, wb[...], preferred_element_type=jnp.float32) + bb[...]


def _proj2(xv, wa, ba, wb, bb):
    blk = 1000
    return pl.pallas_call(
        _proj2_body,
        grid=(NN // blk,),
        in_specs=[
            pl.BlockSpec((blk, DD), lambda i: (i, 0)),
            pl.BlockSpec((DD, DM), lambda i: (0, 0)),
            pl.BlockSpec((1, DM), lambda i: (0, 0)),
            pl.BlockSpec((DD, DM), lambda i: (0, 0)),
            pl.BlockSpec((1, DM), lambda i: (0, 0)),
        ],
        out_specs=[pl.BlockSpec((blk, DM), lambda i: (i, 0))] * 2,
        out_shape=[jax.ShapeDtypeStruct((NN, DM), jnp.float32)] * 2,
    )(xv, wa, ba, wb, bb)


def _proj1_body(y_ref, wa, ba, oa):
    oa[...] = (jnp.dot(y_ref[...], wa[...], preferred_element_type=jnp.float32)
               + ba[...])


def _proj1(yv, wa, ba):
    blk = 2000
    return pl.pallas_call(
        _proj1_body,
        grid=(NE // blk,),
        in_specs=[
            pl.BlockSpec((blk, DD), lambda i: (i, 0)),
            pl.BlockSpec((DD, DM), lambda i: (0, 0)),
            pl.BlockSpec((1, DM), lambda i: (0, 0)),
        ],
        out_specs=pl.BlockSpec((blk, DM), lambda i: (i, 0)),
        out_shape=jax.ShapeDtypeStruct((NE, DM), jnp.float32),
    )(yv, wa, ba)


def _ffn_body(f_ref, w1, b1, w2, b2, o_ref):
    ft = f_ref[0] + f_ref[1]
    h = jnp.dot(ft, w1[...], preferred_element_type=jnp.float32) + b1[...]
    h = h * (1.0 / (1.0 + jnp.exp(-h)))
    o_ref[...] = (jnp.dot(h, w2[...], preferred_element_type=jnp.float32)
                  + b2[...])


def _ffn(ftn, w1, b1, w2, b2):
    blk = 1000
    return pl.pallas_call(
        _ffn_body,
        grid=(NN // blk,),
        in_specs=[
            pl.BlockSpec((NC, blk, DM), lambda i: (0, i, 0)),
            pl.BlockSpec((DM, HH), lambda i: (0, 0)),
            pl.BlockSpec((1, HH), lambda i: (0, 0)),
            pl.BlockSpec((HH, DD), lambda i: (0, 0)),
            pl.BlockSpec((1, DD), lambda i: (0, 0)),
        ],
        out_specs=pl.BlockSpec((blk, DD), lambda i: (i, 0)),
        out_shape=jax.ShapeDtypeStruct((NN, DD), jnp.float32),
    )(ftn, w1, b1, w2, b2)


def kernel(x, y, r, edge_index, t_edge_index, W_src, b_src, W_dst, b_dst,
           W_edge, b_edge, attn, W1, b1, W2, b2):
    f32 = jnp.float32
    srcv = edge_index[0].astype(jnp.int32)
    dstv = edge_index[1].astype(jnp.int32)
    tsv = t_edge_index[0].astype(jnp.int32)
    tdv = t_edge_index[1].astype(jnp.int32)
    xj, xi = _proj2(x.astype(f32), W_src.astype(f32),
                    b_src.reshape(1, DM).astype(f32),
                    W_dst.astype(f32), b_dst.reshape(1, DM).astype(f32))
    yp = _proj1(y.astype(f32), W_edge.astype(f32),
                b_edge.reshape(1, DM).astype(f32))
    xr = _build(xj, xi, yp, srcv, dstv, r.astype(f32).reshape(-1))
    xr = xr.reshape(NE, XRW)
    a, tmax = _score(xr, tsv, tdv, attn.reshape(4, 16).astype(f32))
    aexp, part = _denom(a, tdv, tmax)
    ainv = _inv(part)
    ftn = _msg(xr, tsv, tdv, aexp, ainv, dstv)
    return _ffn(ftn, W1.astype(f32), b1.reshape(1, HH).astype(f32),
                W2.astype(f32), b2.reshape(1, DD).astype(f32))


# speedup vs baseline: 39.2580x; 1.9329x over previous
import functools

import jax
import jax.numpy as jnp
from jax import lax
from jax.experimental import pallas as pl
from jax.experimental.pallas import tpu as pltpu
from jax.experimental.pallas import tpu_sc as plsc

NN = 10000
NE = 160000
NT = 640000
DD = 256
DM = 64
HH = 1024
XRW = 80

NC, NS, LL = 2, 16, 16
NW = NC * NS
BT = 80
NBT = NT // (NW * BT)
BE = 80
NBE = NE // BE

_mesh = plsc.VectorSubcoreMesh(
    core_axis_name="c", subcore_axis_name="s", num_cores=NC, num_subcores=NS)
_sc_params = pltpu.CompilerParams(needs_layout_passes=False,
                                  use_tc_tiling_on_sc=False)


def _worker_id():
    return lax.axis_index("s") * NC + lax.axis_index("c")


def _rsqrt3(n2):
    i = plsc.bitcast(n2, jnp.int32)
    y = plsc.bitcast(jnp.int32(0x5F3759DF) - (i >> 1), jnp.float32)
    for _ in range(3):
        y = y * (1.5 - 0.5 * n2 * y * y)
    return y


def _silu(v):
    return v / (1.0 + jnp.exp(-v))


def _build_body(xj, xi, yp, srcv, dstv, rv, xr_out,
                idx_s0, idx_s1, idx_d0, idx_d1, ypb0, ypb1, rb0, rb1,
                gx0, gx1, gy0, gy1, xrb0, xrb1,
                ss0, ss1, sd0, sd1, sy0, sy1, sv0, sv1,
                sgx0, sgx1, sgy0, sgy1, sw0, sw1):
    wid = _worker_id()
    iota = lax.iota(jnp.int32, 16)
    zero16 = jnp.zeros((16,), jnp.float32)
    idx_s = (idx_s0, idx_s1)
    idx_d = (idx_d0, idx_d1)
    ypb = (ypb0, ypb1)
    rb = (rb0, rb1)
    gx = (gx0, gx1)
    gy = (gy0, gy1)
    xrb = (xrb0, xrb1)
    ss = (ss0, ss1)
    sd = (sd0, sd1)
    sy = (sy0, sy1)
    sv = (sv0, sv1)
    sgx = (sgx0, sgx1)
    sgy = (sgy0, sgy1)
    sw = (sw0, sw1)

    for b in range(2):
        def _zpad(t, _, b=b):
            xrb[b][pl.ds(t * XRW + DM, 16)] = zero16
            return 0
        lax.fori_loop(0, BE, _zpad, 0)

    def _issue1(i, b):
        base = (wid + NW * i) * BE
        pltpu.async_copy(srcv.at[pl.ds(base, BE)], idx_s[b], ss[b])
        pltpu.async_copy(dstv.at[pl.ds(base, BE)], idx_d[b], sd[b])
        pltpu.async_copy(yp.at[pl.ds(base, BE)], ypb[b], sy[b])
        pltpu.async_copy(rv.at[pl.ds(base * 3, BE * 3)],
                         rb[b].at[pl.ds(0, BE * 3)], sv[b])

    def _wait1(b):
        pltpu.make_async_copy(srcv.at[pl.ds(0, BE)], idx_s[b], ss[b]).wait()
        pltpu.make_async_copy(dstv.at[pl.ds(0, BE)], idx_d[b], sd[b]).wait()
        pltpu.make_async_copy(yp.at[pl.ds(0, BE)], ypb[b], sy[b]).wait()
        pltpu.make_async_copy(rv.at[pl.ds(0, BE * 3)],
                              rb[b].at[pl.ds(0, BE * 3)], sv[b]).wait()

    def _body(i, b, last):
        _wait1(b)
        pltpu.async_copy(xj.at[idx_s[b]], gx[b], sgx[b])
        pltpu.async_copy(xi.at[idx_d[b]], gy[b], sgy[b])
        if not last:
            @pl.when((i + 1 < 62) | (wid < NBE - 62 * NW))
            def _():
                _issue1(i + 1, b ^ 1)
        pltpu.make_async_copy(xj.at[idx_s[b]], gx[b], sgx[b]).wait()
        pltpu.make_async_copy(xi.at[idx_d[b]], gy[b], sgy[b]).wait()

        @pl.when(i >= 2)
        def _():
            pltpu.make_async_copy(
                xrb[b], xr_out.at[pl.ds(0, BE * XRW)], sw[b]).wait()

        def _row(t, _2):
            for k in range(4):
                sl = pl.ds(k * 16, 16)
                xrb[b][pl.ds(t * XRW + k * 16, 16)] = (
                    gx[b][t, sl] + gy[b][t, sl] + ypb[b][t, sl])
            return 0
        lax.fori_loop(0, BE, _row, 0)

        for g in range(BE // 16):
            rows3 = (g * 16 + iota) * 3
            rows80 = (g * 16 + iota) * XRW
            rx = plsc.load_gather(rb[b], [rows3])
            ry = plsc.load_gather(rb[b], [rows3 + 1])
            rz = plsc.load_gather(rb[b], [rows3 + 2])
            inv = _rsqrt3(rx * rx + ry * ry + rz * rz)
            plsc.store_scatter(xrb[b], [rows80 + DM], rx * inv)
            plsc.store_scatter(xrb[b], [rows80 + DM + 1], ry * inv)
            plsc.store_scatter(xrb[b], [rows80 + DM + 2], rz * inv)
        pltpu.async_copy(
            xrb[b], xr_out.at[pl.ds((wid + NW * i) * BE * XRW, BE * XRW)],
            sw[b])

    _issue1(0, 0)

    def _pair(i2, _):
        for b in range(2):
            _body(i2 * 2 + b, b, last=False)
        return 0
    lax.fori_loop(0, 31, _pair, 0)

    @pl.when(wid < NBE - 62 * NW)
    def _():
        _body(62, 0, last=True)
    for b in range(2):
        pltpu.make_async_copy(
            xrb[b], xr_out.at[pl.ds(0, BE * XRW)], sw[b]).wait()


_build = pl.kernel(
    _build_body,
    out_type=jax.ShapeDtypeStruct((NE * XRW,), jnp.float32),
    mesh=_mesh,
    compiler_params=_sc_params,
    scratch_types=(
        [pltpu.VMEM((BE,), jnp.int32)] * 4
        + [pltpu.VMEM((BE, DM), jnp.float32)] * 2
        + [pltpu.VMEM((256,), jnp.float32)] * 2
        + [pltpu.VMEM((BE, DM), jnp.float32)] * 4
        + [pltpu.VMEM((BE * XRW,), jnp.float32)] * 2
        + [pltpu.SemaphoreType.DMA] * 14
    ),
)


def _score_body(xr, tsv, tdv, attnrep, a_out, tmax_out,
                itsa, itda, rts0, rtd0, rts1, rtd1, aba, atr, tmb,
                s10, s20, s11, s21):
    wid = _worker_id()
    iota = lax.iota(jnp.int32, 16)
    ones = jnp.ones((16,), jnp.float32)
    pltpu.sync_copy(attnrep, atr)
    ninf = jnp.full((16,), -3.0e38, jnp.float32)
    tpw = NT // NW
    wb = wid * tpw
    pltpu.sync_copy(tsv.at[pl.ds(wb, tpw)], itsa)
    pltpu.sync_copy(tdv.at[pl.ds(wb, tpw)], itda)

    rts = (rts0, rts1)
    rtd = (rtd0, rtd1)
    s1 = (s10, s11)
    s2 = (s20, s21)

    def _issue(j, b):
        pltpu.async_copy(xr.at[itsa.at[pl.ds(j * BT, BT)]], rts[b], s1[b])
        pltpu.async_copy(xr.at[itda.at[pl.ds(j * BT, BT)]], rtd[b], s2[b])

    def _compute(i, rtsb, rtdb, mv):
        def _grp(g, mv):
            rows = g * 16 + iota
            cc = lambda j: jnp.full((16,), j, jnp.int32)
            cos = (plsc.load_gather(rtsb, [rows, cc(DM)])
                   * plsc.load_gather(rtdb, [rows, cc(DM)])
                   + plsc.load_gather(rtsb, [rows, cc(DM + 1)])
                   * plsc.load_gather(rtdb, [rows, cc(DM + 1)])
                   + plsc.load_gather(rtsb, [rows, cc(DM + 2)])
                   * plsc.load_gather(rtdb, [rows, cc(DM + 2)]))
            two = cos + cos
            x = (plsc.load_gather(rtsb, [rows, cc(0)])
                 + plsc.load_gather(rtdb, [rows, cc(0)]))
            acc = _silu(ones + x) * atr[pl.ds(0, 16)]
            x = (plsc.load_gather(rtsb, [rows, cc(1)])
                 + plsc.load_gather(rtdb, [rows, cc(1)]))
            acc = acc + _silu(cos + x) * atr[pl.ds(16, 16)]

            def _ch(c, carry):
                z0, z1, acc = carry
                colv = lax.broadcast(c, (16,))
                x = (plsc.load_gather(rtsb, [rows, colv])
                     + plsc.load_gather(rtdb, [rows, colv]))
                z = two * z1 - z0
                acc = acc + _silu(z + x) * atr[pl.ds(c * 16, 16)]
                return (z1, z, acc)
            _, _, acc = lax.fori_loop(2, DM, _ch, (ones, cos, acc), unroll=8)
            aba[pl.ds(i * BT + g * 16, 16)] = acc
            return jnp.maximum(mv, acc)
        return lax.fori_loop(0, BT // 16, _grp, mv)

    _issue(0, 0)

    def _blk2(i2, mv):
        for b in range(2):
            i = i2 * 2 + b
            pltpu.make_async_copy(
                xr.at[itsa.at[pl.ds(0, BT)]], rts[b], s1[b]).wait()
            pltpu.make_async_copy(
                xr.at[itda.at[pl.ds(0, BT)]], rtd[b], s2[b]).wait()

            @pl.when(i + 1 < NBT)
            def _():
                _issue(i + 1, b ^ 1)
            mv = _compute(i, rts[b], rtd[b], mv)
        return mv
    mv = lax.fori_loop(0, NBT // 2, _blk2, ninf)
    pltpu.sync_copy(aba, a_out.at[pl.ds(wb, tpw)])
    tmb[...] = mv
    pltpu.sync_copy(tmb, tmax_out.at[wid])


_score = pl.kernel(
    _score_body,
    out_type=(jax.ShapeDtypeStruct((NT,), jnp.float32),
              jax.ShapeDtypeStruct((NW, 16), jnp.float32)),
    mesh=_mesh,
    compiler_params=_sc_params,
    scratch_types=[
        pltpu.VMEM((NT // NW,), jnp.int32), pltpu.VMEM((NT // NW,), jnp.int32),
        pltpu.VMEM((BT, XRW), jnp.float32), pltpu.VMEM((BT, XRW), jnp.float32),
        pltpu.VMEM((BT, XRW), jnp.float32), pltpu.VMEM((BT, XRW), jnp.float32),
        pltpu.VMEM((NT // NW,), jnp.float32), pltpu.VMEM((DM * 16,), jnp.float32),
        pltpu.VMEM((16,), jnp.float32),
        pltpu.SemaphoreType.DMA, pltpu.SemaphoreType.DMA,
        pltpu.SemaphoreType.DMA, pltpu.SemaphoreType.DMA,
    ],
)


def _denom_body(av, tdv, tmaxv, aexp_out, part_out,
                aba, itda, idc0, idc1, tmb, zb2, expb, so, sc0, sc1, asum_sh):
    cid = lax.axis_index("c")
    sid = lax.axis_index("s")
    wid = _worker_id()
    tpw = NT // NW
    wb = wid * tpw
    pltpu.sync_copy(tmaxv, tmb)
    mv = tmb[0]
    for w in range(1, NW):
        mv = jnp.maximum(mv, tmb[w])
    gm = jnp.max(mv)
    zero16 = jnp.zeros((16,), jnp.float32)
    idc = (idc0, idc1)
    sc = (sc0, sc1)

    def _z(j, _):
        zb2[pl.ds(j * 16, 16)] = zero16
        return 0
    lax.fori_loop(0, 2000 // 16, _z, 0)
    for q in range(5):
        pltpu.sync_copy(zb2, asum_sh.at[pl.ds(sid * 10000 + q * 2000, 2000)])
    plsc.subcore_barrier()

    pltpu.sync_copy(av.at[pl.ds(wb, tpw)], aba)
    pltpu.sync_copy(tdv.at[pl.ds(wb, tpw)], itda)

    def _e(j, _):
        sl = pl.ds(j * 16, 16)
        aba[sl] = jnp.exp(aba[sl] - gm)
        return 0
    lax.fori_loop(0, tpw // 16, _e, 0, unroll=8)
    do = pltpu.async_copy(aba, aexp_out.at[pl.ds(wb, tpw)], so)

    def _blk2(i2, _):
        for b in range(2):
            i = i2 * 2 + b

            @pl.when(i >= 2)
            def _():
                pltpu.make_async_copy(
                    aba.at[pl.ds(0, BT)], asum_sh.at[idc[b]], sc[b]).wait()
            for g in range(BT // 16):
                sl = pl.ds(g * 16, 16)
                idc[b][sl] = itda[pl.ds(i * BT + g * 16, 16)]
            pltpu.async_copy(
                aba.at[pl.ds(i * BT, BT)], asum_sh.at[idc[b]], sc[b],
                add=True)
        return 0
    lax.fori_loop(0, NBT // 2, _blk2, 0)
    for b in range(2):
        pltpu.make_async_copy(
            aba.at[pl.ds(0, BT)], asum_sh.at[idc[b]], sc[b]).wait()
    do.wait()
    plsc.subcore_barrier()
    pltpu.sync_copy(asum_sh.at[pl.ds(sid * 10000, 10000)], expb)
    pltpu.sync_copy(expb, part_out.at[cid, pl.ds(sid * 10000, 10000)])


_denom = pl.kernel(
    _denom_body,
    out_type=(jax.ShapeDtypeStruct((NT,), jnp.float32),
              jax.ShapeDtypeStruct((NC, NE), jnp.float32)),
    mesh=_mesh,
    compiler_params=_sc_params,
    scratch_types=[
        pltpu.VMEM((NT // NW,), jnp.float32), pltpu.VMEM((NT // NW,), jnp.int32),
        pltpu.VMEM((BT,), jnp.int32), pltpu.VMEM((BT,), jnp.int32),
        pltpu.VMEM((NW, 16), jnp.float32), pltpu.VMEM((2000,), jnp.float32),
        pltpu.VMEM((10000,), jnp.float32),
        pltpu.SemaphoreType.DMA, pltpu.SemaphoreType.DMA,
        pltpu.SemaphoreType.DMA,
        pltpu.VMEM_SHARED((NE,), jnp.float32),
    ],
)


def _inv_body(partv, ainv_out, b0, b1):
    wid = _worker_id()

    def _blk(i, _):
        b = wid + NW * i

        @pl.when(b < NBE)
        def _():
            base = b * BE
            pltpu.sync_copy(partv.at[0, pl.ds(base, BE)], b0)
            pltpu.sync_copy(partv.at[1, pl.ds(base, BE)], b1)
            for g in range(BE // 16):
                sl = pl.ds(g * 16, 16)
                b0[sl] = 1.0 / (b0[sl] + b1[sl] + 1e-9)
            pltpu.sync_copy(b0, ainv_out.at[pl.ds(base, BE)])
        return 0
    lax.fori_loop(0, (NBE + NW - 1) // NW, _blk, 0)


_inv = pl.kernel(
    _inv_body,
    out_type=jax.ShapeDtypeStruct((NE,), jnp.float32),
    mesh=_mesh,
    compiler_params=_sc_params,
    scratch_types=[
        pltpu.VMEM((BE,), jnp.float32), pltpu.VMEM((BE,), jnp.float32),
    ],
)


def _msg_body(xr, tsv, tdv, aev, ainvv, dstv, ftn_out,
              itsa, itda, aeb0, aeb1, rts0, rts1, avb0, avb1, ndb0, ndb1,
              nds0, nds1, msg0, msg1, expb,
              sr0, sr1, sa0, sa1, sn0, sn1, sm0, sm1, sv0, sv1, ftn_sh):
    cid = lax.axis_index("c")
    sid = lax.axis_index("s")
    wid = _worker_id()
    tpw = NT // NW
    wb = wid * tpw
    rpt = NN // NS
    zero16 = jnp.zeros((16,), jnp.float32)
    rts = (rts0, rts1)
    aeb = (aeb0, aeb1)
    sv = (sv0, sv1)
    avb = (avb0, avb1)
    ndb = (ndb0, ndb1)
    nds = (nds0, nds1)
    msg = (msg0, msg1)
    sr = (sr0, sr1)
    sa = (sa0, sa1)
    sn = (sn0, sn1)
    sm = (sm0, sm1)

    def _z(t, _):
        for k in range(4):
            expb[t, pl.ds(k * 16, 16)] = zero16
        return 0
    lax.fori_loop(0, 125, _z, 0)
    for q in range(5):
        pltpu.sync_copy(expb, ftn_sh.at[pl.ds(sid * rpt + q * 125, 125)])
    plsc.subcore_barrier()

    pltpu.sync_copy(tsv.at[pl.ds(wb, tpw)], itsa)
    pltpu.sync_copy(tdv.at[pl.ds(wb, tpw)], itda)

    def _issue(j, b):
        pltpu.async_copy(xr.at[itsa.at[pl.ds(j * BT, BT)]], rts[b], sr[b])
        pltpu.async_copy(ainvv.at[itda.at[pl.ds(j * BT, BT)]], avb[b], sa[b])
        pltpu.async_copy(dstv.at[itda.at[pl.ds(j * BT, BT)]], ndb[b], sn[b])
        pltpu.async_copy(aev.at[pl.ds(wb + j * BT, BT)], aeb[b], sv[b])

    _issue(0, 0)

    def _blk2(i2, _):
        for b in range(2):
            i = i2 * 2 + b
            pltpu.make_async_copy(
                xr.at[itsa.at[pl.ds(0, BT)]], rts[b], sr[b]).wait()
            pltpu.make_async_copy(
                ainvv.at[itda.at[pl.ds(0, BT)]], avb[b], sa[b]).wait()
            pltpu.make_async_copy(
                dstv.at[itda.at[pl.ds(0, BT)]], ndb[b], sn[b]).wait()
            pltpu.make_async_copy(
                aev.at[pl.ds(0, BT)], aeb[b], sv[b]).wait()

            @pl.when(i + 1 < NBT)
            def _():
                _issue(i + 1, b ^ 1)

            @pl.when(i >= 2)
            def _():
                pltpu.make_async_copy(
                    msg[b], ftn_sh.at[nds[b]], sm[b]).wait()

            def _grp(g, _2):
                sl = pl.ds(g * 16, 16)
                w16 = aeb[b][sl] * avb[b][sl]
                nds[b][sl] = ndb[b][sl]
                for k2 in range(16):
                    t = g * 16 + k2
                    wt = lax.broadcast(w16[k2], (16,))
                    for k in range(4):
                        slk = pl.ds(k * 16, 16)
                        msg[b][t, slk] = rts[b][t, slk] * wt
                return 0
            lax.fori_loop(0, BT // 16, _grp, 0)
            pltpu.async_copy(msg[b], ftn_sh.at[nds[b]], sm[b], add=True)
        return 0
    lax.fori_loop(0, NBT // 2, _blk2, 0)
    for b in range(2):
        pltpu.make_async_copy(msg[b], ftn_sh.at[nds[b]], sm[b]).wait()
    plsc.subcore_barrier()
    for q in range(5):
        pltpu.sync_copy(ftn_sh.at[pl.ds(sid * rpt + q * 125, 125)], expb)
        pltpu.sync_copy(expb, ftn_out.at[cid, pl.ds(sid * rpt + q * 125, 125)])


_msg = pl.kernel(
    _msg_body,
    out_type=jax.ShapeDtypeStruct((NC, NN, DM), jnp.float32),
    mesh=_mesh,
    compiler_params=_sc_params,
    scratch_types=[
        pltpu.VMEM((NT // NW,), jnp.int32), pltpu.VMEM((NT // NW,), jnp.int32),
        pltpu.VMEM((BT,), jnp.float32), pltpu.VMEM((BT,), jnp.float32),
        pltpu.VMEM((BT, XRW), jnp.float32), pltpu.VMEM((BT, XRW), jnp.float32),
        pltpu.VMEM((BT,), jnp.float32), pltpu.VMEM((BT,), jnp.float32),
        pltpu.VMEM((BT,), jnp.int32), pltpu.VMEM((BT,), jnp.int32),
        pltpu.VMEM((BT,), jnp.int32), pltpu.VMEM((BT,), jnp.int32),
        pltpu.VMEM((BT, DM), jnp.float32), pltpu.VMEM((BT, DM), jnp.float32),
        pltpu.VMEM((125, DM), jnp.float32),
        pltpu.SemaphoreType.DMA, pltpu.SemaphoreType.DMA,
        pltpu.SemaphoreType.DMA, pltpu.SemaphoreType.DMA,
        pltpu.SemaphoreType.DMA, pltpu.SemaphoreType.DMA,
        pltpu.SemaphoreType.DMA, pltpu.SemaphoreType.DMA,
        pltpu.SemaphoreType.DMA, pltpu.SemaphoreType.DMA,
        pltpu.VMEM_SHARED((NN, DM), jnp.float32),
    ],
)


def _proj2_body(x_ref, wa, ba, wb, bb, oa, ob):
    xv = x_ref[...]
    oa[...] = jnp.dot(xv, wa[...], preferred_element_type=jnp.float32) + ba[...]
    ob[...] = jnp.dot(xv, wb[...], preferred_element_type=jnp.float32) + bb[...]


def _proj2(xv, wa, ba, wb, bb):
    blk = 1000
    return pl.pallas_call(
        _proj2_body,
        grid=(NN // blk,),
        in_specs=[
            pl.BlockSpec((blk, DD), lambda i: (i, 0)),
            pl.BlockSpec((DD, DM), lambda i: (0, 0)),
            pl.BlockSpec((1, DM), lambda i: (0, 0)),
            pl.BlockSpec((DD, DM), lambda i: (0, 0)),
            pl.BlockSpec((1, DM), lambda i: (0, 0)),
        ],
        out_specs=[pl.BlockSpec((blk, DM), lambda i: (i, 0))] * 2,
        out_shape=[jax.ShapeDtypeStruct((NN, DM), jnp.float32)] * 2,
    )(xv, wa, ba, wb, bb)


def _proj1_body(y_ref, wa, ba, oa):
    oa[...] = (jnp.dot(y_ref[...], wa[...], preferred_element_type=jnp.float32)
               + ba[...])


def _proj1(yv, wa, ba):
    blk = 2000
    return pl.pallas_call(
        _proj1_body,
        grid=(NE // blk,),
        in_specs=[
            pl.BlockSpec((blk, DD), lambda i: (i, 0)),
            pl.BlockSpec((DD, DM), lambda i: (0, 0)),
            pl.BlockSpec((1, DM), lambda i: (0, 0)),
        ],
        out_specs=pl.BlockSpec((blk, DM), lambda i: (i, 0)),
        out_shape=jax.ShapeDtypeStruct((NE, DM), jnp.float32),
    )(yv, wa, ba)


def _ffn_body(f_ref, w1, b1, w2, b2, o_ref):
    ft = f_ref[0] + f_ref[1]
    h = jnp.dot(ft, w1[...], preferred_element_type=jnp.float32) + b1[...]
    h = h * (1.0 / (1.0 + jnp.exp(-h)))
    o_ref[...] = (jnp.dot(h, w2[...], preferred_element_type=jnp.float32)
                  + b2[...])


def _ffn(ftn, w1, b1, w2, b2):
    blk = 1000
    return pl.pallas_call(
        _ffn_body,
        grid=(NN // blk,),
        in_specs=[
            pl.BlockSpec((NC, blk, DM), lambda i: (0, i, 0)),
            pl.BlockSpec((DM, HH), lambda i: (0, 0)),
            pl.BlockSpec((1, HH), lambda i: (0, 0)),
            pl.BlockSpec((HH, DD), lambda i: (0, 0)),
            pl.BlockSpec((1, DD), lambda i: (0, 0)),
        ],
        out_specs=pl.BlockSpec((blk, DD), lambda i: (i, 0)),
        out_shape=jax.ShapeDtypeStruct((NN, DD), jnp.float32),
    )(ftn, w1, b1, w2, b2)


def kernel(x, y, r, edge_index, t_edge_index, W_src, b_src, W_dst, b_dst,
           W_edge, b_edge, attn, W1, b1, W2, b2):
    f32 = jnp.float32
    srcv = edge_index[0].astype(jnp.int32)
    dstv = edge_index[1].astype(jnp.int32)
    tsv = t_edge_index[0].astype(jnp.int32)
    tdv = t_edge_index[1].astype(jnp.int32)
    xj, xi = _proj2(x.astype(f32), W_src.astype(f32),
                    b_src.reshape(1, DM).astype(f32),
                    W_dst.astype(f32), b_dst.reshape(1, DM).astype(f32))
    yp = _proj1(y.astype(f32), W_edge.astype(f32),
                b_edge.reshape(1, DM).astype(f32))
    xr = _build(xj, xi, yp, srcv, dstv, r.astype(f32).reshape(-1))
    xr = xr.reshape(NE, XRW)
    arep = jnp.broadcast_to(attn.astype(f32).reshape(DM, 1), (DM, 16)).reshape(-1)
    a, tmax = _score(xr, tsv, tdv, arep)
    aexp, part = _denom(a, tdv, tmax)
    ainv = _inv(part)
    ftn = _msg(xr, tsv, tdv, aexp, ainv, dstv)
    return _ffn(ftn, W1.astype(f32), b1.reshape(1, HH).astype(f32),
                W2.astype(f32), b2.reshape(1, DD).astype(f32))
